# baseline (speedup 1.0000x reference)
import jax
import jax.numpy as jnp
from jax.experimental import pallas as pl
from jax.experimental.pallas import tpu as pltpu

E = 16
DIM = 1024
INTER = 512
T = 32
A = 2


def _moe_ffn_kernel(idx_ref, x_ref, w1_ref, w2_ref, out0_ref, out1_ref):
    e = pl.program_id(0)
    x = x_ref[...]
    h = jnp.dot(x, w1_ref[0], preferred_element_type=jnp.float32)
    gate = h[:, :INTER]
    up = h[:, INTER:]
    act = (gate * jax.nn.sigmoid(gate)) * up
    out_e = jnp.dot(act, w2_ref[0], preferred_element_type=jnp.float32)

    mask = idx_ref[...] == e
    c0 = jnp.where(mask[:, 0:1], out_e, 0.0)
    c1 = jnp.where(mask[:, 1:2], out_e, 0.0)

    @pl.when(e == 0)
    def _init():
        out0_ref[...] = c0
        out1_ref[...] = c1

    @pl.when(e != 0)
    def _accum():
        out0_ref[...] += c0
        out1_ref[...] += c1


def kernel(x, expert_indices, w1, w2):
    expert_indices = expert_indices.astype(jnp.int32)
    out0, out1 = pl.pallas_call(
        _moe_ffn_kernel,
        grid=(E,),
        in_specs=[
            pl.BlockSpec((T, A), lambda e: (0, 0)),
            pl.BlockSpec((T, DIM), lambda e: (0, 0)),
            pl.BlockSpec((1, DIM, 2 * INTER), lambda e: (e, 0, 0)),
            pl.BlockSpec((1, INTER, DIM), lambda e: (e, 0, 0)),
        ],
        out_specs=[
            pl.BlockSpec((T, DIM), lambda e: (0, 0)),
            pl.BlockSpec((T, DIM), lambda e: (0, 0)),
        ],
        out_shape=[
            jax.ShapeDtypeStruct((T, DIM), jnp.float32),
            jax.ShapeDtypeStruct((T, DIM), jnp.float32),
        ],
        compiler_params=pltpu.CompilerParams(vmem_limit_bytes=100 * 1024 * 1024),
    )(expert_indices, x, w1, w2)
    return jnp.stack([out0, out1], axis=1)
